# baseline (speedup 1.0000x reference)
import functools

import jax
import jax.numpy as jnp
from jax import lax
from jax.experimental import pallas as pl
from jax.experimental.pallas import tpu as pltpu
from jax.experimental.pallas import tpu_sc as plsc

B = 1024
T = 200
DIM = 128
VOCAB = 1000

NC = 2
NS = 16
L = 16
NW = NC * NS

N_ROWS = B * T
ROWS_PER_W = N_ROWS // NW
CHUNK = 128
N_CHUNKS = ROWS_PER_W // CHUNK
VECS_PER_CHUNK = CHUNK // L
NBUF = 5
AHEAD = 4
N_GROUPS = N_CHUNKS // NBUF
STAGE_ROWS = 64


def _gather_body(idx_hbm, t0_hbm, t1_hbm, t2_hbm, t3_hbm, out_hbm,
                 raw_v, idx_v, table_sp,
                 b0, b1, b2, b3, b4,
                 g0, g1, g2, g3, g4, w0, w1, w2, w3, w4, tsem):
    bufs = [b0, b1, b2, b3, b4]
    gsems = [g0, g1, g2, g3, g4]
    wsems = [w0, w1, w2, w3, w4]
    tables = [t0_hbm, t1_hbm, t2_hbm, t3_hbm]

    sid = lax.axis_index("s")
    wid = sid * NC + lax.axis_index("c")
    base = wid * ROWS_PER_W

    @pl.when(sid < NS - 1)
    def _():
        for q, t in enumerate(tables):
            pltpu.async_copy(
                t.at[pl.ds(sid * STAGE_ROWS, STAGE_ROWS)],
                table_sp.at[pl.ds(q * VOCAB + sid * STAGE_ROWS, STAGE_ROWS)],
                tsem)

    @pl.when(sid == NS - 1)
    def _():
        rem = VOCAB - (NS - 1) * STAGE_ROWS
        for q, t in enumerate(tables):
            pltpu.async_copy(
                t.at[pl.ds((NS - 1) * STAGE_ROWS, rem)],
                table_sp.at[pl.ds(q * VOCAB + (NS - 1) * STAGE_ROWS, rem)],
                tsem)

    pltpu.sync_copy(idx_hbm.at[pl.ds(base, ROWS_PER_W)], raw_v)
    off = (lax.iota(jnp.int32, L) % 4) * VOCAB

    def idx_body(k, carry):
        for j in range(VECS_PER_CHUNK):
            idx_v[k, pl.ds(j * L, L)] = (
                raw_v[pl.ds(k * CHUNK + j * L, L)] + off)
        return carry

    lax.fori_loop(0, N_CHUNKS, idx_body, 0, unroll=False)

    @pl.when(sid < NS - 1)
    def _():
        for q in range(4):
            pltpu.make_async_copy(
                tables[q].at[pl.ds(0, STAGE_ROWS)],
                table_sp.at[pl.ds(0, STAGE_ROWS)], tsem).wait()

    @pl.when(sid == NS - 1)
    def _():
        rem = VOCAB - (NS - 1) * STAGE_ROWS
        for q in range(4):
            pltpu.make_async_copy(
                tables[q].at[pl.ds(0, rem)],
                table_sp.at[pl.ds(0, rem)], tsem).wait()

    plsc.subcore_barrier()

    def fire_gather(k, p):
        pltpu.async_copy(table_sp.at[idx_v.at[k % N_CHUNKS]], bufs[p],
                         gsems[p])

    def wait_gather(k, p):
        pltpu.make_async_copy(table_sp.at[idx_v.at[k % N_CHUNKS]], bufs[p],
                              gsems[p]).wait()

    def fire_write(k, p):
        pltpu.async_copy(bufs[p],
                         out_hbm.at[pl.ds(base + (k % N_CHUNKS) * CHUNK,
                                          CHUNK)], wsems[p])

    def wait_write(p):
        pltpu.make_async_copy(bufs[p], out_hbm.at[pl.ds(base, CHUNK)],
                              wsems[p]).wait()

    for k in range(AHEAD):
        fire_gather(k, k)

    def group_body(g, carry):
        for p in range(NBUF):
            k = g * NBUF + p
            fp = (p + AHEAD) % NBUF

            wait_gather(k, p)
            fire_write(k, p)

            @pl.when(jnp.logical_and(k + AHEAD >= NBUF,
                                     k + AHEAD < N_CHUNKS))
            def _():
                wait_write(fp)

            @pl.when(k + AHEAD < N_CHUNKS)
            def _():
                fire_gather(k + AHEAD, fp)
        return carry

    lax.fori_loop(0, N_GROUPS, group_body, 0, unroll=False)

    for p in range(NBUF):
        wait_write(p)


@jax.jit
def _rvq_embed(idx_flat, t0, t1, t2, t3):
    mesh = plsc.VectorSubcoreMesh(core_axis_name="c", subcore_axis_name="s")
    run = functools.partial(
        pl.kernel,
        out_type=jax.ShapeDtypeStruct((N_ROWS, DIM), jnp.float32),
        mesh=mesh,
        scratch_types=[
            pltpu.VMEM((ROWS_PER_W,), jnp.int32),
            pltpu.VMEM((N_CHUNKS, CHUNK), jnp.int32),
            pltpu.VMEM_SHARED((4 * VOCAB, DIM), jnp.float32),
        ] + [pltpu.VMEM((CHUNK, DIM), jnp.float32) for _ in range(NBUF)]
          + [pltpu.SemaphoreType.DMA for _ in range(2 * NBUF + 1)],
    )(_gather_body)
    return run(idx_flat, t0, t1, t2, t3)


def kernel(x, W0, W1, W2, W3):
    idx_flat = x.reshape(-1).astype(jnp.int32)
    out = _rvq_embed(idx_flat, W3, W0, W1, W2)
    return out.reshape(B, T, DIM)


# speedup vs baseline: 1.0021x; 1.0021x over previous
import functools

import jax
import jax.numpy as jnp
from jax import lax
from jax.experimental import pallas as pl
from jax.experimental.pallas import tpu as pltpu
from jax.experimental.pallas import tpu_sc as plsc

B = 1024
T = 200
DIM = 128
VOCAB = 1000

NC = 2
NS = 16
L = 16
NW = NC * NS

N_ROWS = B * T
ROWS_PER_W = N_ROWS // NW
CHUNK = 128
N_CHUNKS = ROWS_PER_W // CHUNK
VECS_PER_CHUNK = CHUNK // L
NBUF = 5
AHEAD = 3
N_GROUPS = N_CHUNKS // NBUF
STAGE_ROWS = 64


def _gather_body(idx_hbm, t0_hbm, t1_hbm, t2_hbm, t3_hbm, out_hbm,
                 raw_v, idx_v, table_sp,
                 b0, b1, b2, b3, b4,
                 g0, g1, g2, g3, g4, w0, w1, w2, w3, w4, tsem):
    bufs = [b0, b1, b2, b3, b4]
    gsems = [g0, g1, g2, g3, g4]
    wsems = [w0, w1, w2, w3, w4]
    tables = [t0_hbm, t1_hbm, t2_hbm, t3_hbm]

    sid = lax.axis_index("s")
    wid = sid * NC + lax.axis_index("c")
    base = wid * ROWS_PER_W

    @pl.when(sid < NS - 1)
    def _():
        for q, t in enumerate(tables):
            pltpu.async_copy(
                t.at[pl.ds(sid * STAGE_ROWS, STAGE_ROWS)],
                table_sp.at[pl.ds(q * VOCAB + sid * STAGE_ROWS, STAGE_ROWS)],
                tsem)

    @pl.when(sid == NS - 1)
    def _():
        rem = VOCAB - (NS - 1) * STAGE_ROWS
        for q, t in enumerate(tables):
            pltpu.async_copy(
                t.at[pl.ds((NS - 1) * STAGE_ROWS, rem)],
                table_sp.at[pl.ds(q * VOCAB + (NS - 1) * STAGE_ROWS, rem)],
                tsem)

    pltpu.sync_copy(idx_hbm.at[pl.ds(base, ROWS_PER_W)], raw_v)
    off = (lax.iota(jnp.int32, L) % 4) * VOCAB

    def idx_body(k, carry):
        for j in range(VECS_PER_CHUNK):
            idx_v[k, pl.ds(j * L, L)] = (
                raw_v[pl.ds(k * CHUNK + j * L, L)] + off)
        return carry

    lax.fori_loop(0, N_CHUNKS, idx_body, 0, unroll=False)

    @pl.when(sid < NS - 1)
    def _():
        for q in range(4):
            pltpu.make_async_copy(
                tables[q].at[pl.ds(0, STAGE_ROWS)],
                table_sp.at[pl.ds(0, STAGE_ROWS)], tsem).wait()

    @pl.when(sid == NS - 1)
    def _():
        rem = VOCAB - (NS - 1) * STAGE_ROWS
        for q in range(4):
            pltpu.make_async_copy(
                tables[q].at[pl.ds(0, rem)],
                table_sp.at[pl.ds(0, rem)], tsem).wait()

    plsc.subcore_barrier()

    def fire_gather(k, p):
        pltpu.async_copy(table_sp.at[idx_v.at[k % N_CHUNKS]], bufs[p],
                         gsems[p])

    def wait_gather(k, p):
        pltpu.make_async_copy(table_sp.at[idx_v.at[k % N_CHUNKS]], bufs[p],
                              gsems[p]).wait()

    def fire_write(k, p):
        pltpu.async_copy(bufs[p],
                         out_hbm.at[pl.ds(base + (k % N_CHUNKS) * CHUNK,
                                          CHUNK)], wsems[p])

    def wait_write(p):
        pltpu.make_async_copy(bufs[p], out_hbm.at[pl.ds(base, CHUNK)],
                              wsems[p]).wait()

    for k in range(AHEAD):
        fire_gather(k, k)

    def group_body(g, carry):
        for p in range(NBUF):
            k = g * NBUF + p
            fp = (p + AHEAD) % NBUF

            wait_gather(k, p)
            fire_write(k, p)

            @pl.when(jnp.logical_and(k + AHEAD >= NBUF,
                                     k + AHEAD < N_CHUNKS))
            def _():
                wait_write(fp)

            @pl.when(k + AHEAD < N_CHUNKS)
            def _():
                fire_gather(k + AHEAD, fp)
        return carry

    lax.fori_loop(0, N_GROUPS, group_body, 0, unroll=False)

    for p in range(NBUF):
        wait_write(p)


@jax.jit
def _rvq_embed(idx_flat, t0, t1, t2, t3):
    mesh = plsc.VectorSubcoreMesh(core_axis_name="c", subcore_axis_name="s")
    run = functools.partial(
        pl.kernel,
        out_type=jax.ShapeDtypeStruct((N_ROWS, DIM), jnp.float32),
        mesh=mesh,
        scratch_types=[
            pltpu.VMEM((ROWS_PER_W,), jnp.int32),
            pltpu.VMEM((N_CHUNKS, CHUNK), jnp.int32),
            pltpu.VMEM_SHARED((4 * VOCAB, DIM), jnp.float32),
        ] + [pltpu.VMEM((CHUNK, DIM), jnp.float32) for _ in range(NBUF)]
          + [pltpu.SemaphoreType.DMA for _ in range(2 * NBUF + 1)],
    )(_gather_body)
    return run(idx_flat, t0, t1, t2, t3)


def kernel(x, W0, W1, W2, W3):
    idx_flat = x.reshape(-1).astype(jnp.int32)
    out = _rvq_embed(idx_flat, W3, W0, W1, W2)
    return out.reshape(B, T, DIM)
